# Initial kernel scaffold; baseline (speedup 1.0000x reference)
import functools

import jax
import jax.numpy as jnp
from jax import lax
from jax.experimental import pallas as pl
from jax.experimental.pallas import tpu as pltpu
from jax.experimental.pallas import tpu_sc as plsc

N = 10000
D = 128
E = 320000
NCLS = 40

NC = 2
NS = 16
NW = NC * NS
L = 16

EPT = E // NW
C = 80
NWIN = EPT // C
RPT = N // NS


def _encode(x, W_enc, b_enc, W1):
    def body(x_ref, we_ref, be_ref, w1_ref, o_ref):
        h = jnp.tanh(
            jnp.dot(x_ref[...], we_ref[...], preferred_element_type=jnp.float32)
            + be_ref[...]
        )
        o_ref[...] = jnp.dot(h, w1_ref[...], preferred_element_type=jnp.float32)

    return pl.pallas_call(
        body,
        out_shape=jax.ShapeDtypeStruct((N, D), jnp.float32),
    )(x, W_enc, b_enc.reshape(1, D), W1)


def _mid(parts, b, W):
    def body(p_ref, b_ref, w_ref, o_ref):
        h = jnp.tanh(p_ref[0] + p_ref[1] + b_ref[...])
        o_ref[...] = jnp.dot(h, w_ref[...], preferred_element_type=jnp.float32)

    return pl.pallas_call(
        body,
        out_shape=jax.ShapeDtypeStruct((N, D), jnp.float32),
    )(parts, b.reshape(1, D), W)


def _decode(parts, b2, W_dec, b_dec):
    def body(p_ref, b2_ref, wd_ref, bd_ref, o_ref):
        h = jnp.tanh(p_ref[0] + p_ref[1] + b2_ref[...])
        o_ref[...] = (
            jnp.dot(h, wd_ref[...], preferred_element_type=jnp.float32)
            + bd_ref[...]
        )

    return pl.pallas_call(
        body,
        out_shape=jax.ShapeDtypeStruct((N, NCLS), jnp.float32),
    )(parts, b2.reshape(1, D), W_dec, b_dec.reshape(1, NCLS))


def _sc_edge_pass(hw, src, dst, ew, zeros):
    mesh = plsc.VectorSubcoreMesh(core_axis_name="c", subcore_axis_name="s")

    @functools.partial(
        pl.kernel,
        mesh=mesh,
        out_type=jax.ShapeDtypeStruct((NC, N, D), jnp.float32),
        scratch_types=[
            pltpu.VMEM((C,), jnp.int32),
            pltpu.VMEM((C,), jnp.int32),
            pltpu.VMEM((C,), jnp.float32),
            pltpu.VMEM((C, D), jnp.float32),
            pltpu.VMEM_SHARED((N, D), jnp.float32),
            pltpu.SemaphoreType.DMA,
        ],
    )
    def k(hw_hbm, src_hbm, dst_hbm, ew_hbm, z_hbm, out_hbm,
          src_v, dst_v, ew_v, rows_v, acc_sh, sem):
        cid = lax.axis_index("c")
        sid = lax.axis_index("s")

        pltpu.sync_copy(z_hbm.at[pl.ds(sid * RPT, RPT)],
                        acc_sh.at[pl.ds(sid * RPT, RPT)])
        plsc.subcore_barrier()

        ebase = (cid * NS + sid) * EPT

        @pl.loop(0, NWIN)
        def _(w):
            base = ebase + w * C
            pltpu.sync_copy(src_hbm.at[pl.ds(base, C)], src_v)
            pltpu.sync_copy(dst_hbm.at[pl.ds(base, C)], dst_v)
            pltpu.sync_copy(ew_hbm.at[pl.ds(base, C)], ew_v)
            pltpu.async_copy(hw_hbm.at[src_v], rows_v, sem).wait()

            @pl.loop(0, C)
            def _(r):
                widx = jnp.full((L,), r, jnp.int32)
                wvec = plsc.load_gather(ew_v, [widx])
                for cc in range(D // L):
                    sl = pl.ds(cc * L, L)
                    rows_v[r, sl] = rows_v[r, sl] * wvec

            pltpu.sync_copy(rows_v, acc_sh.at[dst_v], add=True)

        plsc.subcore_barrier()
        pltpu.sync_copy(acc_sh.at[pl.ds(sid * RPT, RPT)],
                        out_hbm.at[cid, pl.ds(sid * RPT, RPT)])

    return k(hw, src, dst, ew, zeros)


def kernel(x, edge_index, edge_weight, W_enc, b_enc, W1, b1, W2, b2, W_dec, b_dec):
    src = edge_index[0].astype(jnp.int32)
    dst = edge_index[1].astype(jnp.int32)
    ew = edge_weight.astype(jnp.float32)
    zeros = jnp.zeros((N, D), jnp.float32)

    hw1 = _encode(x, W_enc, b_enc, W1)
    p1 = _sc_edge_pass(hw1, src, dst, ew, zeros)
    hw2 = _mid(p1, b1, W2)
    p2 = _sc_edge_pass(hw2, src, dst, ew, zeros)
    return _decode(p2, b2, W_dec, b_dec)


# speedup vs baseline: 3.4313x; 3.4313x over previous
import dataclasses
import functools

import jax
import jax.numpy as jnp
from jax import lax
from jax.experimental import pallas as pl
from jax.experimental.pallas import tpu as pltpu
from jax.experimental.pallas import tpu_sc as plsc

N = 10000
D = 128
E = 320000
NCLS = 40

NC = 2
NS = 16
NW = NC * NS
L = 16

EPT = E // NW
C = 80
NWIN = EPT // C
NPAD = 10240
RPT = NPAD // NS


def _encode(x, W_enc, b_enc, W1):
    def body(x_ref, we_ref, be_ref, w1_ref, o_ref):
        h = jnp.tanh(
            jnp.dot(x_ref[...], we_ref[...], preferred_element_type=jnp.float32)
            + be_ref[...]
        )
        o_ref[...] = jnp.dot(h, w1_ref[...], preferred_element_type=jnp.float32)

    return pl.pallas_call(
        body,
        out_shape=jax.ShapeDtypeStruct((N, D), jnp.float32),
    )(x, W_enc, b_enc.reshape(1, D), W1)


def _mid(parts, b, W):
    def body(p_ref, b_ref, w_ref, o_ref):
        h = jnp.tanh(p_ref[0, :N, :] + p_ref[1, :N, :] + b_ref[...])
        o_ref[...] = jnp.dot(h, w_ref[...], preferred_element_type=jnp.float32)

    return pl.pallas_call(
        body,
        out_shape=jax.ShapeDtypeStruct((N, D), jnp.float32),
    )(parts, b.reshape(1, D), W)


def _decode(parts, b2, W_dec, b_dec):
    def body(p_ref, b2_ref, wd_ref, bd_ref, o_ref):
        h = jnp.tanh(p_ref[0, :N, :] + p_ref[1, :N, :] + b2_ref[...])
        o_ref[...] = (
            jnp.dot(h, wd_ref[...], preferred_element_type=jnp.float32)
            + bd_ref[...]
        )

    return pl.pallas_call(
        body,
        out_shape=jax.ShapeDtypeStruct((N, NCLS), jnp.float32),
    )(parts, b2.reshape(1, D), W_dec, b_dec.reshape(1, NCLS))


def _sc_edge_pass(hw, src, dst, ew, zeros):
    mesh = plsc.VectorSubcoreMesh(core_axis_name="c", subcore_axis_name="s")
    cp = pltpu.CompilerParams()
    if "needs_layout_passes" in pltpu.CompilerParams.__dataclass_fields__:
        cp = dataclasses.replace(cp, needs_layout_passes=False)

    @functools.partial(
        pl.kernel,
        mesh=mesh,
        compiler_params=cp,
        out_type=jax.ShapeDtypeStruct((NC, NPAD, D), jnp.float32),
        scratch_types=[
            pltpu.VMEM((C,), jnp.int32),
            pltpu.VMEM((C,), jnp.int32),
            pltpu.VMEM((C,), jnp.float32),
            pltpu.VMEM((C, D), jnp.float32),
            pltpu.VMEM_SHARED((NPAD, D), jnp.float32),
            pltpu.SemaphoreType.DMA,
        ],
    )
    def k(hw_hbm, src_hbm, dst_hbm, ew_hbm, z_hbm, out_hbm,
          src_v, dst_v, ew_v, rows_v, acc_sh, sem):
        cid = lax.axis_index("c")
        sid = lax.axis_index("s")

        pltpu.sync_copy(z_hbm.at[pl.ds(sid * RPT, RPT)],
                        acc_sh.at[pl.ds(sid * RPT, RPT)])
        plsc.subcore_barrier()

        ebase = (cid * NS + sid) * EPT

        @pl.loop(0, NWIN)
        def _(w):
            base = ebase + w * C
            pltpu.sync_copy(src_hbm.at[pl.ds(base, C)], src_v)
            pltpu.sync_copy(dst_hbm.at[pl.ds(base, C)], dst_v)
            pltpu.sync_copy(ew_hbm.at[pl.ds(base, C)], ew_v)
            pltpu.async_copy(hw_hbm.at[src_v], rows_v, sem).wait()

            @pl.loop(0, C)
            def _(r):
                widx = jnp.full((L,), r, jnp.int32)
                wvec = plsc.load_gather(ew_v, [widx])
                for cc in range(D // L):
                    sl = pl.ds(cc * L, L)
                    rows_v[r, sl] = rows_v[r, sl] * wvec

            pltpu.sync_copy(rows_v, acc_sh.at[dst_v], add=True)

        plsc.subcore_barrier()
        pltpu.sync_copy(acc_sh.at[pl.ds(sid * RPT, RPT)],
                        out_hbm.at[cid, pl.ds(sid * RPT, RPT)])

    return k(hw, src, dst, ew, zeros)


def kernel(x, edge_index, edge_weight, W_enc, b_enc, W1, b1, W2, b2, W_dec, b_dec):
    src = edge_index[0].astype(jnp.int32)
    dst = edge_index[1].astype(jnp.int32)
    ew = edge_weight.astype(jnp.float32)
    zeros = jnp.zeros((NPAD, D), jnp.float32)

    hw1 = _encode(x, W_enc, b_enc, W1)
    p1 = _sc_edge_pass(hw1, src, dst, ew, zeros)
    hw2 = _mid(p1, b1, W2)
    p2 = _sc_edge_pass(hw2, src, dst, ew, zeros)
    return _decode(p2, b2, W_dec, b_dec)
